# Initial kernel scaffold; baseline (speedup 1.0000x reference)
import jax
import jax.numpy as jnp
from jax.experimental import pallas as pl


def kernel(x, edge_index, edge_weight, W1, b1, W2, b2):
    raise NotImplementedError("write your pallas kernel here")


# speedup vs baseline: 4.9594x; 4.9594x over previous
import functools

import jax
import jax.numpy as jnp
from jax import lax
from jax.experimental import pallas as pl
from jax.experimental.pallas import tpu as pltpu
from jax.experimental.pallas import tpu_sc as plsc

N = 10000
E = 160000
INDIM = 256
HIDIM = 128
OUTDIM = 256

NC, NS, L = 2, 16, 16
E_PAD = 163840
EPT_ALL = E_PAD // NS
EPC = E_PAD // NC
EPT = EPC // NS
BLK = 128
N_PAD = 10240
NPT = N_PAD // NS
FH = 64

_MESH = plsc.VectorSubcoreMesh(
    core_axis_name="c", subcore_axis_name="s", num_cores=NC, num_subcores=NS)
_SC_PARAMS = pltpu.CompilerParams(
    needs_layout_passes=False, use_tc_tiling_on_sc=False)


def _zero_gbuf(gbuf):
    z = jnp.zeros((16,), jnp.float32)

    def body(i, _):
        r = i // (FH // 16)
        j = i % (FH // 16)
        gbuf[r, pl.ds(j * 16, 16)] = z
        return 0

    lax.fori_loop(0, BLK * (FH // 16), body, 0)


def _newton_dis(deg_v, dis_v):
    def body(i, _):
        d = deg_v[pl.ds(i * 16, 16)] + 1.0
        ib = plsc.bitcast(d, jnp.int32)
        y = plsc.bitcast(0x5F3759DF - (ib >> 1), jnp.float32)
        for _ in range(3):
            y = y * (1.5 - 0.5 * d * y * y)
        dis_v[pl.ds(i * 16, 16)] = y
        return 0

    lax.fori_loop(0, N_PAD // 16, body, 0)


def _msg_loop(src_hbm, row_v, col_v, norm_v, gbuf, rblk, cblk, acc, sem,
              n_blocks):
    def blk_body(b, _):
        for j in range(BLK // 16):
            rblk[pl.ds(j * 16, 16)] = row_v[pl.ds(b * BLK + j * 16, 16)]
            cblk[pl.ds(j * 16, 16)] = col_v[pl.ds(b * BLK + j * 16, 16)]
        pltpu.async_copy(src_hbm.at[rblk], gbuf, sem).wait()

        def scale_e(e, _):
            ns = plsc.load_gather(
                norm_v, [jnp.full((16,), b * BLK + e, jnp.int32)])
            for j in range(FH // 16):
                v = gbuf[e, pl.ds(j * 16, 16)]
                gbuf[e, pl.ds(j * 16, 16)] = v * ns
            return 0

        lax.fori_loop(0, BLK, scale_e, 0)
        pltpu.sync_copy(gbuf, acc.at[cblk], add=True)
        return 0

    lax.fori_loop(0, n_blocks, blk_body, 0)


def _zero_acc(gbuf, acc, s):
    _zero_gbuf(gbuf)
    for k in range(NPT // BLK):
        pltpu.sync_copy(gbuf, acc.at[pl.ds(s * NPT + k * BLK, BLK)])


def _copy_out(acc, out_hbm, s):
    pltpu.sync_copy(acc.at[pl.ds(s * NPT, NPT)],
                    out_hbm.at[pl.ds(s * NPT, NPT)])


_P64 = jax.ShapeDtypeStruct((N_PAD, FH), jnp.float32)


@functools.partial(
    pl.kernel,
    out_type=(
        jax.ShapeDtypeStruct((E_PAD,), jnp.float32),
        jax.ShapeDtypeStruct((N,), jnp.float32),
        _P64, _P64,
        _P64, _P64,
    ),
    mesh=_MESH,
    scratch_types=[
        pltpu.VMEM((EPT_ALL,), jnp.int32),
        pltpu.VMEM((EPT_ALL,), jnp.float32),
        pltpu.VMEM((EPT,), jnp.int32),
        pltpu.VMEM((EPT,), jnp.int32),
        pltpu.VMEM((EPT,), jnp.float32),
        pltpu.VMEM((EPT,), jnp.float32),
        pltpu.VMEM((N_PAD,), jnp.float32),
        pltpu.VMEM((N_PAD,), jnp.float32),
        pltpu.VMEM((BLK, FH), jnp.float32),
        pltpu.VMEM((BLK,), jnp.int32),
        pltpu.VMEM((BLK,), jnp.int32),
        pltpu.VMEM((BLK,), jnp.float32),
        pltpu.VMEM_SHARED((N_PAD,), jnp.float32),
        pltpu.VMEM_SHARED((N_PAD, FH), jnp.float32),
        pltpu.SemaphoreType.DMA,
    ],
    compiler_params=_SC_PARAMS,
)
def _sc_layer1(row_hbm, col_hbm, ew_hbm, h1a_hbm, h1b_hbm,
               norm_hbm, dis_hbm, p0a_hbm, p0b_hbm, p1a_hbm, p1b_hbm,
               col_d, ew_d, row_m, col_m, ew_m, norm_m, deg_v, dis_v,
               gbuf, rblk, cblk, wblk, deg_sp, acc, sem):
    c = lax.axis_index("c")
    s = lax.axis_index("s")

    pltpu.sync_copy(col_hbm.at[pl.ds(s * EPT_ALL, EPT_ALL)], col_d)
    pltpu.sync_copy(ew_hbm.at[pl.ds(s * EPT_ALL, EPT_ALL)], ew_d)
    mbase = c * EPC + s * EPT
    pltpu.sync_copy(row_hbm.at[pl.ds(mbase, EPT)], row_m)
    pltpu.sync_copy(col_hbm.at[pl.ds(mbase, EPT)], col_m)
    pltpu.sync_copy(ew_hbm.at[pl.ds(mbase, EPT)], ew_m)

    z = jnp.zeros((16,), jnp.float32)
    for j in range(BLK // 16):
        wblk[pl.ds(j * 16, 16)] = z
    for k in range(NPT // BLK):
        pltpu.sync_copy(wblk, deg_sp.at[pl.ds(s * NPT + k * BLK, BLK)])
    plsc.subcore_barrier()

    def deg_blk(b, _):
        for j in range(BLK // 16):
            cblk[pl.ds(j * 16, 16)] = col_d[pl.ds(b * BLK + j * 16, 16)]
            wblk[pl.ds(j * 16, 16)] = ew_d[pl.ds(b * BLK + j * 16, 16)]
        pltpu.sync_copy(wblk, deg_sp.at[cblk], add=True)
        return 0

    lax.fori_loop(0, EPT_ALL // BLK, deg_blk, 0)
    plsc.subcore_barrier()

    pltpu.sync_copy(deg_sp, deg_v)
    _newton_dis(deg_v, dis_v)

    @pl.when(jnp.logical_and(c == 0, s == 0))
    def _():
        pltpu.sync_copy(dis_v.at[pl.ds(0, N)], dis_hbm)

    def norm_blk(i, _):
        r = row_m[pl.ds(i * 16, 16)]
        cc = col_m[pl.ds(i * 16, 16)]
        w = ew_m[pl.ds(i * 16, 16)]
        dr = plsc.load_gather(dis_v, [r])
        dc = plsc.load_gather(dis_v, [cc])
        norm_m[pl.ds(i * 16, 16)] = dr * w * dc
        return 0

    lax.fori_loop(0, EPT // 16, norm_blk, 0)
    pltpu.sync_copy(norm_m, norm_hbm.at[pl.ds(mbase, EPT)])

    for p, (src, out0, out1) in enumerate(
            ((h1a_hbm, p0a_hbm, p1a_hbm), (h1b_hbm, p0b_hbm, p1b_hbm))):
        _zero_acc(gbuf, acc, s)
        plsc.subcore_barrier()
        _msg_loop(src, row_m, col_m, norm_m, gbuf, rblk, cblk, acc, sem,
                  EPT // BLK)
        plsc.subcore_barrier()

        @pl.when(c == 0)
        def _(out0=out0):
            _copy_out(acc, out0, s)

        @pl.when(c == 1)
        def _(out1=out1):
            _copy_out(acc, out1, s)


@functools.partial(
    pl.kernel,
    out_type=(_P64, _P64, _P64, _P64),
    mesh=_MESH,
    scratch_types=[
        pltpu.VMEM((EPT_ALL,), jnp.int32),
        pltpu.VMEM((EPT_ALL,), jnp.int32),
        pltpu.VMEM((EPT_ALL,), jnp.float32),
        pltpu.VMEM((BLK, FH), jnp.float32),
        pltpu.VMEM((BLK,), jnp.int32),
        pltpu.VMEM((BLK,), jnp.int32),
        pltpu.VMEM_SHARED((N_PAD, FH), jnp.float32),
        pltpu.SemaphoreType.DMA,
    ],
    compiler_params=_SC_PARAMS,
)
def _sc_layer2(row_hbm, col_hbm, norm_hbm, h2s0_hbm, h2s1_hbm, h2s2_hbm,
               h2s3_hbm, q0_hbm, q1_hbm, q2_hbm, q3_hbm,
               row_m, col_m, norm_m, gbuf, rblk, cblk, acc, sem):
    c = lax.axis_index("c")
    s = lax.axis_index("s")

    base = s * EPT_ALL
    pltpu.sync_copy(row_hbm.at[pl.ds(base, EPT_ALL)], row_m)
    pltpu.sync_copy(col_hbm.at[pl.ds(base, EPT_ALL)], col_m)
    pltpu.sync_copy(norm_hbm.at[pl.ds(base, EPT_ALL)], norm_m)

    for p, (src0, src1, out0, out1) in enumerate(
            ((h2s0_hbm, h2s2_hbm, q0_hbm, q2_hbm),
             (h2s1_hbm, h2s3_hbm, q1_hbm, q3_hbm))):
        _zero_acc(gbuf, acc, s)
        plsc.subcore_barrier()

        @pl.when(c == 0)
        def _(src0=src0):
            _msg_loop(src0, row_m, col_m, norm_m, gbuf, rblk, cblk, acc,
                      sem, EPT_ALL // BLK)

        @pl.when(c == 1)
        def _(src1=src1):
            _msg_loop(src1, row_m, col_m, norm_m, gbuf, rblk, cblk, acc,
                      sem, EPT_ALL // BLK)

        plsc.subcore_barrier()

        @pl.when(c == 0)
        def _(out0=out0):
            _copy_out(acc, out0, s)

        @pl.when(c == 1)
        def _(out1=out1):
            _copy_out(acc, out1, s)


_RB = 1000


def _t1_body(x_ref, w_ref, ha_ref, hb_ref):
    h = jnp.dot(x_ref[...], w_ref[...], preferred_element_type=jnp.float32)
    ha_ref[...] = h[:, :FH]
    hb_ref[...] = h[:, FH:]


def _t2_body(p0a_ref, p0b_ref, p1a_ref, p1b_ref, h1a_ref, h1b_ref,
             dis_ref, b1_ref, w2_ref, s0_ref, s1_ref, s2_ref, s3_ref):
    d2 = dis_ref[...] * dis_ref[...]
    a_left = p0a_ref[...] + p1a_ref[...] + d2 * h1a_ref[...]
    a_right = p0b_ref[...] + p1b_ref[...] + d2 * h1b_ref[...]
    a = jnp.concatenate([a_left, a_right], axis=1) + b1_ref[...]
    a = jnp.maximum(a, 0.0)
    h2 = jnp.dot(a, w2_ref[...], preferred_element_type=jnp.float32)
    s0_ref[...] = h2[:, :FH]
    s1_ref[...] = h2[:, FH:2 * FH]
    s2_ref[...] = h2[:, 2 * FH:3 * FH]
    s3_ref[...] = h2[:, 3 * FH:]


def _t3_body(q0_ref, q1_ref, q2_ref, q3_ref, s0_ref, s1_ref, s2_ref, s3_ref,
             dis_ref, b2_ref, y_ref):
    d2 = dis_ref[...] * dis_ref[...]
    y_ref[...] = jnp.concatenate(
        [q0_ref[...] + d2 * s0_ref[...],
         q1_ref[...] + d2 * s1_ref[...],
         q2_ref[...] + d2 * s2_ref[...],
         q3_ref[...] + d2 * s3_ref[...]], axis=1) + b2_ref[...]


def _row_blk(d):
    return pl.BlockSpec((_RB, d), lambda i: (i, 0))


def _full(shape):
    return pl.BlockSpec(shape, lambda i: tuple(0 for _ in shape))


_F64 = jax.ShapeDtypeStruct((N, FH), jnp.float32)


def kernel(x, edge_index, edge_weight, W1, b1, W2, b2):
    row = edge_index[0]
    col = edge_index[1]
    pad = E_PAD - E
    rowp = jnp.concatenate([row, jnp.zeros((pad,), row.dtype)])
    colp = jnp.concatenate([col, jnp.zeros((pad,), col.dtype)])
    ewp = jnp.concatenate([edge_weight, jnp.zeros((pad,), edge_weight.dtype)])

    h1a, h1b = pl.pallas_call(
        _t1_body,
        grid=(N // _RB,),
        in_specs=[_row_blk(INDIM), _full((INDIM, HIDIM))],
        out_specs=[_row_blk(FH), _row_blk(FH)],
        out_shape=[_F64, _F64],
    )(x, W1)

    norm, dis, p0a, p0b, p1a, p1b = _sc_layer1(rowp, colp, ewp, h1a, h1b)

    dis2d = dis.reshape(N, 1)
    h2s = pl.pallas_call(
        _t2_body,
        grid=(N // _RB,),
        in_specs=[_row_blk(FH)] * 6 +
                 [pl.BlockSpec((_RB, 1), lambda i: (i, 0)),
                  _full((1, HIDIM)), _full((HIDIM, OUTDIM))],
        out_specs=[_row_blk(FH)] * 4,
        out_shape=[_F64] * 4,
    )(p0a, p0b, p1a, p1b, h1a, h1b, dis2d, b1.reshape(1, HIDIM), W2)

    q0, q1, q2, q3 = _sc_layer2(rowp, colp, norm, *h2s)

    y = pl.pallas_call(
        _t3_body,
        grid=(N // _RB,),
        in_specs=[_row_blk(FH)] * 8 +
                 [pl.BlockSpec((_RB, 1), lambda i: (i, 0)),
                  _full((1, OUTDIM))],
        out_specs=_row_blk(OUTDIM),
        out_shape=jax.ShapeDtypeStruct((N, OUTDIM), jnp.float32),
    )(q0, q1, q2, q3, *h2s, dis2d, b2.reshape(1, OUTDIM))

    return y
